# baseline (speedup 1.0000x reference)
import functools

import jax
import jax.numpy as jnp
from jax.experimental import pallas as pl
from jax.experimental.pallas import tpu as pltpu

VOCAB = 128000
EMB = 2048
HID = 10240
EPS = 1e-5
BATCH = 4
SEQ = 2048
NTOK = BATCH * SEQ


def _gather_body(ids_ref, table_blk, out_blk):
    out_blk[...] = table_blk[...]


def _gather_tc(ids_flat, table):
    table3 = table.reshape(VOCAB, 1, EMB)
    grid_spec = pltpu.PrefetchScalarGridSpec(
        num_scalar_prefetch=1,
        grid=(NTOK,),
        in_specs=[
            pl.BlockSpec((1, 1, EMB), lambda i, ids: (ids[i], 0, 0)),
        ],
        out_specs=pl.BlockSpec((1, 1, EMB), lambda i, ids: (i, 0, 0)),
    )
    out = pl.pallas_call(
        _gather_body,
        grid_spec=grid_spec,
        out_shape=jax.ShapeDtypeStruct((NTOK, 1, EMB), jnp.float32),
    )(ids_flat, table3)
    return out.reshape(NTOK, EMB)


LN_BM = 512


def _ln_body(x_ref, g_ref, b_ref, o_ref):
    x = x_ref[...]
    mean = jnp.mean(x, axis=-1, keepdims=True)
    xc = x - mean
    var = jnp.mean(xc * xc, axis=-1, keepdims=True)
    xhat = xc * jax.lax.rsqrt(var + EPS)
    y = xhat * g_ref[...] + b_ref[...]
    o_ref[...] = y.astype(jnp.bfloat16)


def _layernorm_tc(emb, gamma2d, beta2d):
    return pl.pallas_call(
        _ln_body,
        grid=(NTOK // LN_BM,),
        in_specs=[
            pl.BlockSpec((LN_BM, EMB), lambda i: (i, 0)),
            pl.BlockSpec((1, EMB), lambda i: (0, 0)),
            pl.BlockSpec((1, EMB), lambda i: (0, 0)),
        ],
        out_specs=pl.BlockSpec((LN_BM, EMB), lambda i: (i, 0)),
        out_shape=jax.ShapeDtypeStruct((NTOK, EMB), jnp.bfloat16),
    )(emb, gamma2d, beta2d)


MM_BM = 2048
MM_BN = 1024


def _mm_body(h_ref, w_ref, b_ref, o_ref):
    acc = jnp.dot(h_ref[...], w_ref[...], preferred_element_type=jnp.float32)
    o_ref[...] = acc + b_ref[...]


def _matmul_tc(h, w_bf16, bias2d):
    return pl.pallas_call(
        _mm_body,
        grid=(NTOK // MM_BM, HID // MM_BN),
        in_specs=[
            pl.BlockSpec((MM_BM, EMB), lambda m, n: (m, 0)),
            pl.BlockSpec((EMB, MM_BN), lambda m, n: (0, n)),
            pl.BlockSpec((1, MM_BN), lambda m, n: (0, n)),
        ],
        out_specs=pl.BlockSpec((MM_BM, MM_BN), lambda m, n: (m, n)),
        out_shape=jax.ShapeDtypeStruct((NTOK, HID), jnp.float32),
    )(h, w_bf16, bias2d)


@jax.jit
def kernel(input_ids, table, ln_gamma, ln_beta, proj_w, proj_b):
    ids_flat = input_ids.reshape(-1).astype(jnp.int32)
    emb = _gather_tc(ids_flat, table)
    h = _layernorm_tc(emb, ln_gamma.reshape(1, EMB), ln_beta.reshape(1, EMB))
    out = _matmul_tc(h, proj_w.astype(jnp.bfloat16), proj_b.reshape(1, HID))
    return out.reshape(BATCH, SEQ, HID)


# speedup vs baseline: 9.6978x; 9.6978x over previous
import functools

import jax
import jax.numpy as jnp
from jax import lax
from jax.experimental import pallas as pl
from jax.experimental.pallas import tpu as pltpu
from jax.experimental.pallas import tpu_sc as plsc

VOCAB = 128000
EMB = 2048
HID = 10240
EPS = 1e-5
BATCH = 4
SEQ = 2048
NTOK = BATCH * SEQ


def _gather_body(ids_ref, table_blk, out_blk):
    out_blk[...] = table_blk[...]


def _gather_tc(ids_flat, table):
    table3 = table.reshape(VOCAB, 1, EMB)
    grid_spec = pltpu.PrefetchScalarGridSpec(
        num_scalar_prefetch=1,
        grid=(NTOK,),
        in_specs=[
            pl.BlockSpec((1, 1, EMB), lambda i, ids: (ids[i], 0, 0)),
        ],
        out_specs=pl.BlockSpec((1, 1, EMB), lambda i, ids: (i, 0, 0)),
    )
    out = pl.pallas_call(
        _gather_body,
        grid_spec=grid_spec,
        out_shape=jax.ShapeDtypeStruct((NTOK, 1, EMB), jnp.float32),
    )(ids_flat, table3)
    return out.reshape(NTOK, EMB)


_SC_NC = 2
_SC_NS = 16
_SC_NW = _SC_NC * _SC_NS
_ROWS_PER_W = NTOK // _SC_NW
_CH = 16
_NCH = _ROWS_PER_W // _CH


def _gather_sc(ids2d, table):
    mesh = plsc.VectorSubcoreMesh(core_axis_name="c", subcore_axis_name="s")

    @functools.partial(
        pl.kernel,
        mesh=mesh,
        out_type=jax.ShapeDtypeStruct((NTOK, EMB), jnp.float32),
        scratch_types=[
            pltpu.VMEM((_NCH, _CH), jnp.int32),
            pltpu.VMEM((2, _CH, EMB), jnp.float32),
            pltpu.SemaphoreType.DMA,
            pltpu.SemaphoreType.DMA,
            pltpu.SemaphoreType.DMA,
            pltpu.SemaphoreType.DMA,
        ],
    )
    def k(ids_hbm, table_hbm, out_hbm, idx_v, rows_v, gs0, gs1, ws0, ws1):
        wid = lax.axis_index("s") * _SC_NC + lax.axis_index("c")
        base = wid * _ROWS_PER_W
        pltpu.sync_copy(ids_hbm.at[pl.ds(wid * _NCH, _NCH)], idx_v)
        gsem = [gs0, gs1]
        wsem = [ws0, ws1]
        gcp = [None, None]
        wcp = [None, None]

        def start_gather(j):
            b = j % 2
            gcp[b] = pltpu.async_copy(
                table_hbm.at[idx_v.at[j]], rows_v.at[b], gsem[b])

        start_gather(0)
        for j in range(_NCH):
            b = j % 2
            gcp[b].wait()
            if j + 1 < _NCH:
                if j >= 1:
                    wcp[1 - b].wait()
                start_gather(j + 1)
            wcp[b] = pltpu.async_copy(
                rows_v.at[b], out_hbm.at[pl.ds(base + j * _CH, _CH)], wsem[b])
        wcp[_NCH % 2].wait()
        wcp[(_NCH - 1) % 2].wait()

    return k(ids2d, table)


LN_BM = 512


def _ln_body(x_ref, g_ref, b_ref, o_ref):
    x = x_ref[...]
    mean = jnp.mean(x, axis=-1, keepdims=True)
    xc = x - mean
    var = jnp.mean(xc * xc, axis=-1, keepdims=True)
    xhat = xc * jax.lax.rsqrt(var + EPS)
    y = xhat * g_ref[...] + b_ref[...]
    o_ref[...] = y.astype(jnp.bfloat16)


def _layernorm_tc(emb, gamma2d, beta2d):
    return pl.pallas_call(
        _ln_body,
        grid=(NTOK // LN_BM,),
        in_specs=[
            pl.BlockSpec((LN_BM, EMB), lambda i: (i, 0)),
            pl.BlockSpec((1, EMB), lambda i: (0, 0)),
            pl.BlockSpec((1, EMB), lambda i: (0, 0)),
        ],
        out_specs=pl.BlockSpec((LN_BM, EMB), lambda i: (i, 0)),
        out_shape=jax.ShapeDtypeStruct((NTOK, EMB), jnp.bfloat16),
    )(emb, gamma2d, beta2d)


MM_BM = 2048
MM_BN = 1024


def _mm_body(h_ref, w_ref, b_ref, o_ref):
    acc = jnp.dot(h_ref[...], w_ref[...], preferred_element_type=jnp.float32)
    o_ref[...] = acc + b_ref[...]


def _matmul_tc(h, w_bf16, bias2d):
    return pl.pallas_call(
        _mm_body,
        grid=(NTOK // MM_BM, HID // MM_BN),
        in_specs=[
            pl.BlockSpec((MM_BM, EMB), lambda m, n: (m, 0)),
            pl.BlockSpec((EMB, MM_BN), lambda m, n: (0, n)),
            pl.BlockSpec((1, MM_BN), lambda m, n: (0, n)),
        ],
        out_specs=pl.BlockSpec((MM_BM, MM_BN), lambda m, n: (m, n)),
        out_shape=jax.ShapeDtypeStruct((NTOK, HID), jnp.float32),
    )(h, w_bf16, bias2d)


@jax.jit
def kernel(input_ids, table, ln_gamma, ln_beta, proj_w, proj_b):
    ids2d = input_ids.reshape(_SC_NW * _NCH, _CH).astype(jnp.int32)
    emb = _gather_sc(ids2d, table)
    h = _layernorm_tc(emb, ln_gamma.reshape(1, EMB), ln_beta.reshape(1, EMB))
    out = _matmul_tc(h, proj_w.astype(jnp.bfloat16), proj_b.reshape(1, HID))
    return out.reshape(BATCH, SEQ, HID)
